# Initial kernel scaffold; baseline (speedup 1.0000x reference)
import jax
import jax.numpy as jnp
from jax.experimental import pallas as pl


def kernel(x, t, z, edge_index, W_gcn_o, b_gcn_o, W_gcn_t, b_gcn_t, a_src, a_dst, W_enc, b_enc, p1_W1, p1_b1, p1_W2, p1_b2, p1_W3, p1_b3, p0_W1, p0_b1, p0_W2, p0_b2, p0_W3, p0_b3, d_W1, d_b1, d_W2, d_b2, d_W3, d_b3):
    raise NotImplementedError("write your pallas kernel here")


# speedup vs baseline: 38.7911x; 38.7911x over previous
import functools

import jax
import jax.numpy as jnp
from jax import lax
from jax.experimental import pallas as pl
from jax.experimental.pallas import tpu as pltpu
from jax.experimental.pallas import tpu_sc as plsc

N = 10000
H = 128
E = 320000
NC = 2
NS = 16
NW = NC * NS
C = 64
NCHUNK = 160
EPW = NCHUNK * C
EPAD = NW * EPW
NACC = 10240
RPT = NACC // NS

_mesh = plsc.VectorSubcoreMesh(
    core_axis_name="c", subcore_axis_name="s", num_cores=NC, num_subcores=NS)
_scp = pltpu.CompilerParams(needs_layout_passes=False)


@functools.partial(
    pl.kernel,
    out_type=jax.ShapeDtypeStruct((NC * NACC,), jnp.float32),
    mesh=_mesh,
    compiler_params=_scp,
    scratch_types=[
        pltpu.VMEM((EPW,), jnp.int32),
        pltpu.VMEM((16,), jnp.float32),
        pltpu.VMEM((RPT,), jnp.float32),
        pltpu.SemaphoreType.DMA,
        pltpu.VMEM_SHARED((NACC,), jnp.float32),
    ],
)
def _s1_deg(dstE, out, didx1, ones_v, stage_v, sem, deg_sp):
    cid = lax.axis_index("c")
    sid = lax.axis_index("s")
    wid = cid * NS + sid
    ones_v[...] = jnp.ones((16,), jnp.float32)
    row0 = sid * RPT

    def zb(i, _):
        stage_v[pl.ds(i * 16, 16)] = jnp.zeros((16,), jnp.float32)
        return 0
    lax.fori_loop(0, RPT // 16, zb, 0)
    pltpu.sync_copy(dstE.at[wid], didx1)
    pltpu.sync_copy(stage_v, deg_sp.at[pl.ds(row0, RPT)])
    plsc.subcore_barrier()

    def body(g, _):
        dv = didx1[pl.ds(g * 16, 16)]
        pltpu.make_async_copy(ones_v, deg_sp.at[dv], sem).start(add=True)
        return 0
    lax.fori_loop(0, EPW // 16, body, 0)

    def drain(g, _):
        dv0 = didx1[pl.ds(0, 16)]
        pltpu.make_async_copy(ones_v, deg_sp.at[dv0], sem).wait()
        return 0
    lax.fori_loop(0, EPW // 16, drain, 0)
    plsc.subcore_barrier()
    obase = pl.multiple_of(cid * NACC + row0, 8)
    pltpu.sync_copy(deg_sp.at[pl.ds(row0, RPT)], stage_v)
    pltpu.sync_copy(stage_v, out.at[pl.ds(obase, RPT)])


@functools.partial(
    pl.kernel,
    out_type=jax.ShapeDtypeStruct((NC, NACC, H), jnp.float32),
    mesh=_mesh,
    compiler_params=_scp,
    scratch_types=[
        pltpu.VMEM((EPW,), jnp.int32),
        pltpu.VMEM((EPW,), jnp.int32),
        pltpu.VMEM((2, C, H), jnp.float32),
        pltpu.SemaphoreType.DMA,
        pltpu.SemaphoreType.DMA,
        pltpu.VMEM_SHARED((NACC, H), jnp.float32),
    ],
)
def _s2_agg(srcE, dstE, y, out, sidx1, didx1, rows2, g0, g1, acc_sp):
    cid = lax.axis_index("c")
    sid = lax.axis_index("s")
    wid = cid * NS + sid
    row0 = sid * RPT
    gsem = (g0, g1)

    def zb(i, _):
        rows2[0, i // 8, pl.ds((i % 8) * 16, 16)] = jnp.zeros((16,), jnp.float32)
        return 0
    lax.fori_loop(0, C * 8, zb, 0)
    pltpu.sync_copy(srcE.at[wid], sidx1)
    pltpu.sync_copy(dstE.at[wid], didx1)
    for b in range(RPT // C):
        pltpu.sync_copy(rows2.at[0], acc_sp.at[pl.ds(row0 + b * C, C)])
    plsc.subcore_barrier()

    def _gather(k, b, sem):
        o = pl.multiple_of(k * C, 8)
        pltpu.async_copy(y.at[sidx1.at[pl.ds(o, C)]], rows2.at[b], sem)

    def _gwait(b, sem):
        pltpu.make_async_copy(
            y.at[sidx1.at[pl.ds(0, C)]], rows2.at[b], sem).wait()

    for b in range(2):
        _gather(b, b, gsem[b])

    def body(jj, _):
        for b in range(2):
            k = jj * 2 + b
            _gwait(b, gsem[b])
            for g in range(C // 16):
                dv = didx1[pl.ds(k * C + g * 16, 16)]
                pltpu.sync_copy(
                    rows2.at[b, pl.ds(g * 16, 16)], acc_sp.at[dv], add=True)

            @pl.when(jj < NCHUNK // 2 - 1)
            def _():
                _gather(k + 2, b, gsem[b])
        return 0
    lax.fori_loop(0, NCHUNK // 2, body, 0)
    plsc.subcore_barrier()
    for b in range(RPT // C):
        pltpu.sync_copy(acc_sp.at[pl.ds(row0 + b * C, C)], rows2.at[0])
        pltpu.sync_copy(rows2.at[0], out.at[cid, pl.ds(row0 + b * C, C)])


@functools.partial(
    pl.kernel,
    out_type=(
        jax.ShapeDtypeStruct((NC, NACC, H), jnp.float32),
        jax.ShapeDtypeStruct((NC * NACC,), jnp.float32),
    ),
    mesh=_mesh,
    compiler_params=_scp,
    scratch_types=[
        pltpu.VMEM((EPW,), jnp.int32),
        pltpu.VMEM((EPW,), jnp.int32),
        pltpu.VMEM((2, C, H), jnp.float32),
        pltpu.VMEM((2, C), jnp.float32),
        pltpu.VMEM((2, C), jnp.float32),
        pltpu.VMEM((2, C), jnp.float32),
        pltpu.VMEM((16,), jnp.float32),
        pltpu.VMEM((RPT,), jnp.float32),
        pltpu.SemaphoreType.DMA,
        pltpu.SemaphoreType.DMA,
        pltpu.SemaphoreType.DMA,
        pltpu.SemaphoreType.DMA,
        pltpu.VMEM_SHARED((NACC, H), jnp.float32),
        pltpu.VMEM_SHARED((NACC,), jnp.float32),
    ],
)
def _s3_attn(srcE, dstE, rt, sA, sD, gmax, hout, dout,
             sidx1, didx1, rows2, w2, va2, vd2, gmax_v, stage_v,
             ga0, ga1, vs0, vs1,
             hacc_sp, den_sp):
    cid = lax.axis_index("c")
    sid = lax.axis_index("s")
    wid = cid * NS + sid
    row0 = sid * RPT
    gsem = (ga0, ga1)
    vsem = (vs0, vs1)
    pltpu.sync_copy(gmax, gmax_v)
    pltpu.sync_copy(srcE.at[wid], sidx1)
    pltpu.sync_copy(dstE.at[wid], didx1)

    def zb(i, _):
        rows2[0, i // 8, pl.ds((i % 8) * 16, 16)] = jnp.zeros((16,), jnp.float32)
        return 0
    lax.fori_loop(0, C * 8, zb, 0)

    def zb1(i, _):
        stage_v[pl.ds(i * 16, 16)] = jnp.zeros((16,), jnp.float32)
        return 0
    lax.fori_loop(0, RPT // 16, zb1, 0)
    for b in range(RPT // C):
        pltpu.sync_copy(rows2.at[0], hacc_sp.at[pl.ds(row0 + b * C, C)])
    pltpu.sync_copy(stage_v, den_sp.at[pl.ds(row0, RPT)])
    plsc.subcore_barrier()

    def _fetch(k, b):
        o = pl.multiple_of(k * C, 8)
        pltpu.async_copy(sA.at[sidx1.at[pl.ds(o, C)]], va2.at[b], vsem[b])
        pltpu.async_copy(sD.at[didx1.at[pl.ds(o, C)]], vd2.at[b], vsem[b])
        pltpu.async_copy(rt.at[sidx1.at[pl.ds(o, C)]], rows2.at[b], gsem[b])

    for b in range(2):
        _fetch(b, b)
    gm = gmax_v[...]

    def body(jj, _):
        for b in range(2):
            k = jj * 2 + b
            pltpu.make_async_copy(
                sA.at[sidx1.at[pl.ds(0, C)]], va2.at[b], vsem[b]).wait()
            pltpu.make_async_copy(
                sD.at[didx1.at[pl.ds(0, C)]], vd2.at[b], vsem[b]).wait()
            for g in range(C // 16):
                v1 = va2[b, pl.ds(g * 16, 16)]
                v2 = vd2[b, pl.ds(g * 16, 16)]
                es = v1 + v2
                e = jnp.where(es > 0, es, 0.2 * es)
                cc = gm + v2
                cb = jnp.where(cc > 0, cc, 0.2 * cc)
                w2[b, pl.ds(g * 16, 16)] = jnp.exp(e - cb)
            pltpu.make_async_copy(
                rt.at[sidx1.at[pl.ds(0, C)]], rows2.at[b], gsem[b]).wait()

            def rbody(r, _):
                wr = plsc.load_gather(
                    w2, [jnp.full((16,), b, jnp.int32),
                         jnp.full((16,), r, jnp.int32)])
                for g in range(H // 16):
                    rows2[b, r, pl.ds(g * 16, 16)] = (
                        rows2[b, r, pl.ds(g * 16, 16)] * wr)
                return 0
            lax.fori_loop(0, C, rbody, 0)
            for g in range(C // 16):
                dv = didx1[pl.ds(k * C + g * 16, 16)]
                pltpu.sync_copy(
                    w2.at[b, pl.ds(g * 16, 16)], den_sp.at[dv], add=True)
                pltpu.sync_copy(
                    rows2.at[b, pl.ds(g * 16, 16)], hacc_sp.at[dv], add=True)

            @pl.when(jj < NCHUNK // 2 - 1)
            def _():
                _fetch(k + 2, b)
        return 0
    lax.fori_loop(0, NCHUNK // 2, body, 0)
    plsc.subcore_barrier()
    for b in range(RPT // C):
        pltpu.sync_copy(hacc_sp.at[pl.ds(row0 + b * C, C)], rows2.at[0])
        pltpu.sync_copy(rows2.at[0], hout.at[cid, pl.ds(row0 + b * C, C)])
    obase = pl.multiple_of(cid * NACC + row0, 8)
    pltpu.sync_copy(den_sp.at[pl.ds(row0, RPT)], stage_v)
    pltpu.sync_copy(stage_v, dout.at[pl.ds(obase, RPT)])


def _leaky(v):
    return jnp.where(v > 0, v, 0.2 * v)


def _t1_body(deg2_ref, x_ref, y_ref):
    d = deg2_ref[0] + deg2_ref[1] + 1.0
    dinv = (1.0 / jnp.sqrt(d))[:N]
    y_ref[...] = x_ref[...] * dinv


def _t2_body(zp_ref, x_ref, deg2_ref, t_ref, Wo_ref, bo_ref, Wt_ref, bt_ref,
             a1s_ref, a2s_ref, a1d_ref, a2d_ref,
             ro_ref, rt_ref, sA_ref, sD_ref, gmax_ref):
    d = deg2_ref[0] + deg2_ref[1] + 1.0
    dinv = (1.0 / jnp.sqrt(d))[:N]
    zsum = zp_ref[0, :N] + zp_ref[1, :N]
    x = x_ref[...]
    ax = dinv * zsum + (dinv * dinv) * x
    ro = jnp.maximum(
        jnp.dot(ax, Wo_ref[...], preferred_element_type=jnp.float32)
        + bo_ref[...], 0.0)
    rt = jnp.maximum(
        jnp.dot(ax, Wt_ref[...], preferred_element_type=jnp.float32)
        + bt_ref[...], 0.0)
    ro_ref[...] = ro
    rt_ref[...] = rt
    s_src = (jnp.dot(ro, a1s_ref[...], preferred_element_type=jnp.float32)
             + jnp.dot(rt, a2s_ref[...], preferred_element_type=jnp.float32))
    s_dst = (jnp.dot(ro, a1d_ref[...], preferred_element_type=jnp.float32)
             + jnp.dot(rt, a2d_ref[...], preferred_element_type=jnp.float32))
    gmax_ref[...] = jnp.broadcast_to(jnp.max(s_src), (1, H))
    sA_ref[...] = jnp.where(t_ref[...] > 0, s_src, -1e30)
    sD_ref[...] = s_dst


def _mlp(v, W1, b1, W2, b2, W3, b3):
    h1 = _leaky(jnp.dot(v, W1, preferred_element_type=jnp.float32) + b1)
    h2 = _leaky(jnp.dot(h1, W2, preferred_element_type=jnp.float32) + b2)
    return jnp.dot(h2, W3, preferred_element_type=jnp.float32) + b3


def _t3_body(ro_ref, rt_ref, hp_ref, dp_ref, t_ref,
             We1_ref, We2_ref, be_ref,
             p1_refs, p0_refs, d_refs,
             predt_ref, pred_ref, zrep_ref):
    den = dp_ref[0] + dp_ref[1] + 1e-16
    h = (hp_ref[0] + hp_ref[1]) / den
    ro = ro_ref[...]
    rt = rt_ref[...]
    zrep = (jnp.dot(ro, We1_ref[...], preferred_element_type=jnp.float32)
            + jnp.dot(h, We2_ref[...], preferred_element_type=jnp.float32)
            + be_ref[...])
    zrep_ref[...] = zrep
    pred1 = _mlp(zrep, *[r[...] for r in p1_refs])
    pred0 = _mlp(zrep, *[r[...] for r in p0_refs])
    pred_ref[...] = jnp.where(t_ref[...] > 0, pred1, pred0)
    predt_ref[...] = jax.nn.sigmoid(_mlp(rt, *[r[...] for r in d_refs]))


def _t3_flat(ro, rt, hp, dp, t2,
             We1, We2, be,
             p1_W1, p1_b1, p1_W2, p1_b2, p1_W3, p1_b3,
             p0_W1, p0_b1, p0_W2, p0_b2, p0_W3, p0_b3,
             d_W1, d_b1, d_W2, d_b2, d_W3, d_b3,
             predt_ref, pred_ref, zrep_ref):
    _t3_body(ro, rt, hp, dp, t2, We1, We2, be,
             (p1_W1, p1_b1, p1_W2, p1_b2, p1_W3, p1_b3),
             (p0_W1, p0_b1, p0_W2, p0_b2, p0_W3, p0_b3),
             (d_W1, d_b1, d_W2, d_b2, d_W3, d_b3),
             predt_ref, pred_ref, zrep_ref)


def kernel(x, t, z, edge_index, W_gcn_o, b_gcn_o, W_gcn_t, b_gcn_t, a_src,
           a_dst, W_enc, b_enc, p1_W1, p1_b1, p1_W2, p1_b2, p1_W3, p1_b3,
           p0_W1, p0_b1, p0_W2, p0_b2, p0_W3, p0_b3, d_W1, d_b1, d_W2, d_b2,
           d_W3, d_b3):
    f32 = jnp.float32
    src = edge_index[0].astype(jnp.int32)
    dst = edge_index[1].astype(jnp.int32)
    npad = EPAD - E
    srcE = jnp.concatenate(
        [src, jnp.arange(npad, dtype=jnp.int32) % 1024]).reshape(NW, EPW)
    dstE = jnp.concatenate(
        [dst, N + (jnp.arange(npad, dtype=jnp.int32) % (NACC - N))]
    ).reshape(NW, EPW)

    deg_parts = _s1_deg(dstE)
    deg2 = deg_parts.reshape(NC, NACC, 1)

    y = pl.pallas_call(
        _t1_body,
        out_shape=jax.ShapeDtypeStruct((N, H), f32),
    )(deg2, x)

    z_parts = _s2_agg(srcE, dstE, y)

    t2d = t.astype(f32).reshape(N, 1)
    ro, rt, sA2, sD2, gmax2 = pl.pallas_call(
        _t2_body,
        out_shape=(
            jax.ShapeDtypeStruct((N, H), f32),
            jax.ShapeDtypeStruct((N, H), f32),
            jax.ShapeDtypeStruct((N, 1), f32),
            jax.ShapeDtypeStruct((N, 1), f32),
            jax.ShapeDtypeStruct((1, H), f32),
        ),
    )(z_parts, x, deg2, t2d,
      W_gcn_o, b_gcn_o.reshape(1, H), W_gcn_t, b_gcn_t.reshape(1, H),
      a_src[:H].reshape(H, 1), a_src[H:].reshape(H, 1),
      a_dst[:H].reshape(H, 1), a_dst[H:].reshape(H, 1))

    sA = sA2.reshape(N)
    sD = sD2.reshape(N)
    gmax16 = gmax2[0, :16]
    hacc_parts, den_parts = _s3_attn(srcE, dstE, rt, sA, sD, gmax16)

    R = 2048
    _w = pl.BlockSpec((H, H), lambda i: (0, 0))
    _b = pl.BlockSpec((1, H), lambda i: (0, 0))
    _w3 = pl.BlockSpec((H, 1), lambda i: (0, 0))
    _b3 = pl.BlockSpec((1, 1), lambda i: (0, 0))
    _rows = pl.BlockSpec((R, H), lambda i: (i, 0))
    _col = pl.BlockSpec((R, 1), lambda i: (i, 0))
    pred_t, pred, zrep = pl.pallas_call(
        _t3_flat,
        grid=(NACC // R,),
        in_specs=[
            _rows, _rows,
            pl.BlockSpec((NC, R, H), lambda i: (0, i, 0)),
            pl.BlockSpec((NC, R, 1), lambda i: (0, i, 0)),
            _col,
            _w, _w, _b,
            _w, _b, _w, _b, _w3, _b3,
            _w, _b, _w, _b, _w3, _b3,
            _w, _b, _w, _b, _w3, _b3,
        ],
        out_specs=(_col, _col, _rows),
        out_shape=(
            jax.ShapeDtypeStruct((N, 1), f32),
            jax.ShapeDtypeStruct((N, 1), f32),
            jax.ShapeDtypeStruct((N, H), f32),
        ),
    )(ro, rt, hacc_parts, den_parts.reshape(NC, NACC, 1), t2d,
      W_enc[:H], W_enc[H:], b_enc.reshape(1, H),
      p1_W1, p1_b1.reshape(1, H), p1_W2, p1_b2.reshape(1, H),
      p1_W3, p1_b3.reshape(1, 1),
      p0_W1, p0_b1.reshape(1, H), p0_W2, p0_b2.reshape(1, H),
      p0_W3, p0_b3.reshape(1, 1),
      d_W1, d_b1.reshape(1, H), d_W2, d_b2.reshape(1, H),
      d_W3, d_b3.reshape(1, 1))
    return (pred_t, pred, zrep)


# speedup vs baseline: 44.8327x; 1.1557x over previous
import functools

import jax
import jax.numpy as jnp
from jax import lax
from jax.experimental import pallas as pl
from jax.experimental.pallas import tpu as pltpu
from jax.experimental.pallas import tpu_sc as plsc

N = 10000
H = 128
E = 320000
NC = 2
NS = 16
NW = NC * NS
C = 64
NCHUNK = 160
EPW = NCHUNK * C
EPAD = NW * EPW
NACC = 10240
RPT = NACC // NS

_mesh = plsc.VectorSubcoreMesh(
    core_axis_name="c", subcore_axis_name="s", num_cores=NC, num_subcores=NS)
_scp = pltpu.CompilerParams(needs_layout_passes=False)


@functools.partial(
    pl.kernel,
    out_type=jax.ShapeDtypeStruct((NC * NACC,), jnp.float32),
    mesh=_mesh,
    compiler_params=_scp,
    scratch_types=[
        pltpu.VMEM((EPW,), jnp.int32),
        pltpu.VMEM((16,), jnp.float32),
        pltpu.VMEM((RPT,), jnp.float32),
        pltpu.SemaphoreType.DMA,
        pltpu.VMEM_SHARED((NACC,), jnp.float32),
    ],
)
def _s1_deg(dstE, out, didx1, ones_v, stage_v, sem, deg_sp):
    cid = lax.axis_index("c")
    sid = lax.axis_index("s")
    wid = cid * NS + sid
    ones_v[...] = jnp.ones((16,), jnp.float32)
    row0 = sid * RPT

    def zb(i, _):
        stage_v[pl.ds(i * 16, 16)] = jnp.zeros((16,), jnp.float32)
        return 0
    lax.fori_loop(0, RPT // 16, zb, 0)
    pltpu.sync_copy(dstE.at[wid], didx1)
    pltpu.sync_copy(stage_v, deg_sp.at[pl.ds(row0, RPT)])
    plsc.subcore_barrier()

    def body(g, _):
        dv = didx1[pl.ds(g * 16, 16)]
        pltpu.make_async_copy(ones_v, deg_sp.at[dv], sem).start(add=True)
        return 0
    lax.fori_loop(0, EPW // 16, body, 0)

    def drain(g, _):
        dv0 = didx1[pl.ds(0, 16)]
        pltpu.make_async_copy(ones_v, deg_sp.at[dv0], sem).wait()
        return 0
    lax.fori_loop(0, EPW // 16, drain, 0)
    plsc.subcore_barrier()
    obase = pl.multiple_of(cid * NACC + row0, 8)
    pltpu.sync_copy(deg_sp.at[pl.ds(row0, RPT)], stage_v)
    pltpu.sync_copy(stage_v, out.at[pl.ds(obase, RPT)])


@functools.partial(
    pl.kernel,
    out_type=jax.ShapeDtypeStruct((NC, NACC, H), jnp.float32),
    mesh=_mesh,
    compiler_params=_scp,
    scratch_types=[
        pltpu.VMEM((EPW,), jnp.int32),
        pltpu.VMEM((EPW,), jnp.int32),
        pltpu.VMEM((2, C, H), jnp.float32),
        pltpu.SemaphoreType.DMA,
        pltpu.SemaphoreType.DMA,
        pltpu.SemaphoreType.DMA,
        pltpu.SemaphoreType.DMA,
        pltpu.VMEM_SHARED((NACC, H), jnp.float32),
    ],
)
def _s2_agg(srcE, dstE, y, out, sidx1, didx1, rows2, g0, g1, s0, s1, acc_sp):
    cid = lax.axis_index("c")
    sid = lax.axis_index("s")
    wid = cid * NS + sid
    row0 = sid * RPT
    gsem = (g0, g1)
    ssem = (s0, s1)

    def zb(i, _):
        rows2[0, i // 8, pl.ds((i % 8) * 16, 16)] = jnp.zeros((16,), jnp.float32)
        return 0
    lax.fori_loop(0, C * 8, zb, 0)
    pltpu.sync_copy(srcE.at[wid], sidx1)
    pltpu.sync_copy(dstE.at[wid], didx1)
    for b in range(RPT // C):
        pltpu.sync_copy(rows2.at[0], acc_sp.at[pl.ds(row0 + b * C, C)])
    plsc.subcore_barrier()

    def _gather(k, b, sem):
        o = pl.multiple_of(k * C, 8)
        pltpu.async_copy(y.at[sidx1.at[pl.ds(o, C)]], rows2.at[b], sem)

    def _gwait(b, sem):
        pltpu.make_async_copy(
            y.at[sidx1.at[pl.ds(0, C)]], rows2.at[b], sem).wait()

    for b in range(2):
        _gather(b, b, gsem[b])

    def body(jj, _):
        for b in range(2):
            k = jj * 2 + b
            _gwait(b, gsem[b])
            for g in range(C // 16):
                dv = didx1[pl.ds(k * C + g * 16, 16)]
                pltpu.make_async_copy(
                    rows2.at[b, pl.ds(g * 16, 16)], acc_sp.at[dv],
                    ssem[b]).start(add=True)

            @pl.when(jj < NCHUNK // 2 - 1)
            def _():
                for g in range(C // 16):
                    dv0 = didx1[pl.ds(g * 16, 16)]
                    pltpu.make_async_copy(
                        rows2.at[b, pl.ds(g * 16, 16)], acc_sp.at[dv0],
                        ssem[b]).wait()
                _gather(k + 2, b, gsem[b])
        return 0
    lax.fori_loop(0, NCHUNK // 2, body, 0)
    for b in range(2):
        for g in range(C // 16):
            dv0 = didx1[pl.ds(g * 16, 16)]
            pltpu.make_async_copy(
                rows2.at[b, pl.ds(g * 16, 16)], acc_sp.at[dv0],
                ssem[b]).wait()
    plsc.subcore_barrier()
    for b in range(RPT // C):
        pltpu.sync_copy(acc_sp.at[pl.ds(row0 + b * C, C)], rows2.at[0])
        pltpu.sync_copy(rows2.at[0], out.at[cid, pl.ds(row0 + b * C, C)])


@functools.partial(
    pl.kernel,
    out_type=(
        jax.ShapeDtypeStruct((NC, NACC, H), jnp.float32),
        jax.ShapeDtypeStruct((NC * NACC,), jnp.float32),
    ),
    mesh=_mesh,
    compiler_params=_scp,
    scratch_types=[
        pltpu.VMEM((EPW,), jnp.int32),
        pltpu.VMEM((EPW,), jnp.int32),
        pltpu.VMEM((2, C, H), jnp.float32),
        pltpu.VMEM((2, C), jnp.float32),
        pltpu.VMEM((2, C), jnp.float32),
        pltpu.VMEM((2, C), jnp.float32),
        pltpu.VMEM((16,), jnp.float32),
        pltpu.VMEM((RPT,), jnp.float32),
        pltpu.SemaphoreType.DMA,
        pltpu.SemaphoreType.DMA,
        pltpu.SemaphoreType.DMA,
        pltpu.SemaphoreType.DMA,
        pltpu.SemaphoreType.DMA,
        pltpu.SemaphoreType.DMA,
        pltpu.SemaphoreType.DMA,
        pltpu.SemaphoreType.DMA,
        pltpu.VMEM_SHARED((NACC, H), jnp.float32),
        pltpu.VMEM_SHARED((NACC,), jnp.float32),
    ],
)
def _s3_attn(srcE, dstE, rt, sA, sD, gmax, hout, dout,
             sidx1, didx1, rows2, w2, va2, vd2, gmax_v, stage_v,
             ga0, ga1, vs0, vs1, ss0, ss1, ds0, ds1,
             hacc_sp, den_sp):
    cid = lax.axis_index("c")
    sid = lax.axis_index("s")
    wid = cid * NS + sid
    row0 = sid * RPT
    gsem = (ga0, ga1)
    vsem = (vs0, vs1)
    ssem = (ss0, ss1)
    dsem = (ds0, ds1)
    pltpu.sync_copy(gmax, gmax_v)
    pltpu.sync_copy(srcE.at[wid], sidx1)
    pltpu.sync_copy(dstE.at[wid], didx1)

    def zb(i, _):
        rows2[0, i // 8, pl.ds((i % 8) * 16, 16)] = jnp.zeros((16,), jnp.float32)
        return 0
    lax.fori_loop(0, C * 8, zb, 0)

    def zb1(i, _):
        stage_v[pl.ds(i * 16, 16)] = jnp.zeros((16,), jnp.float32)
        return 0
    lax.fori_loop(0, RPT // 16, zb1, 0)
    for b in range(RPT // C):
        pltpu.sync_copy(rows2.at[0], hacc_sp.at[pl.ds(row0 + b * C, C)])
    pltpu.sync_copy(stage_v, den_sp.at[pl.ds(row0, RPT)])
    plsc.subcore_barrier()

    def _fetch(k, b):
        o = pl.multiple_of(k * C, 8)
        pltpu.async_copy(sA.at[sidx1.at[pl.ds(o, C)]], va2.at[b], vsem[b])
        pltpu.async_copy(sD.at[didx1.at[pl.ds(o, C)]], vd2.at[b], vsem[b])
        pltpu.async_copy(rt.at[sidx1.at[pl.ds(o, C)]], rows2.at[b], gsem[b])

    for b in range(2):
        _fetch(b, b)
    gm = gmax_v[...]

    def body(jj, _):
        for b in range(2):
            k = jj * 2 + b
            pltpu.make_async_copy(
                sA.at[sidx1.at[pl.ds(0, C)]], va2.at[b], vsem[b]).wait()
            pltpu.make_async_copy(
                sD.at[didx1.at[pl.ds(0, C)]], vd2.at[b], vsem[b]).wait()
            for g in range(C // 16):
                v1 = va2[b, pl.ds(g * 16, 16)]
                v2 = vd2[b, pl.ds(g * 16, 16)]
                es = v1 + v2
                e = jnp.where(es > 0, es, 0.2 * es)
                cc = gm + v2
                cb = jnp.where(cc > 0, cc, 0.2 * cc)
                w2[b, pl.ds(g * 16, 16)] = jnp.exp(e - cb)
            pltpu.make_async_copy(
                rt.at[sidx1.at[pl.ds(0, C)]], rows2.at[b], gsem[b]).wait()

            def rbody(r, _):
                wr = plsc.load_gather(
                    w2, [jnp.full((16,), b, jnp.int32),
                         jnp.full((16,), r, jnp.int32)])
                for g in range(H // 16):
                    rows2[b, r, pl.ds(g * 16, 16)] = (
                        rows2[b, r, pl.ds(g * 16, 16)] * wr)
                return 0
            lax.fori_loop(0, C, rbody, 0)
            for g in range(C // 16):
                dv = didx1[pl.ds(k * C + g * 16, 16)]
                pltpu.make_async_copy(
                    w2.at[b, pl.ds(g * 16, 16)], den_sp.at[dv],
                    dsem[b]).start(add=True)
                pltpu.make_async_copy(
                    rows2.at[b, pl.ds(g * 16, 16)], hacc_sp.at[dv],
                    ssem[b]).start(add=True)

            @pl.when(jj < NCHUNK // 2 - 1)
            def _():
                for g in range(C // 16):
                    dv0 = didx1[pl.ds(g * 16, 16)]
                    pltpu.make_async_copy(
                        w2.at[b, pl.ds(g * 16, 16)], den_sp.at[dv0],
                        dsem[b]).wait()
                    pltpu.make_async_copy(
                        rows2.at[b, pl.ds(g * 16, 16)], hacc_sp.at[dv0],
                        ssem[b]).wait()
                _fetch(k + 2, b)
        return 0
    lax.fori_loop(0, NCHUNK // 2, body, 0)
    for b in range(2):
        for g in range(C // 16):
            dv0 = didx1[pl.ds(g * 16, 16)]
            pltpu.make_async_copy(
                w2.at[b, pl.ds(g * 16, 16)], den_sp.at[dv0], dsem[b]).wait()
            pltpu.make_async_copy(
                rows2.at[b, pl.ds(g * 16, 16)], hacc_sp.at[dv0],
                ssem[b]).wait()
    plsc.subcore_barrier()
    for b in range(RPT // C):
        pltpu.sync_copy(hacc_sp.at[pl.ds(row0 + b * C, C)], rows2.at[0])
        pltpu.sync_copy(rows2.at[0], hout.at[cid, pl.ds(row0 + b * C, C)])
    obase = pl.multiple_of(cid * NACC + row0, 8)
    pltpu.sync_copy(den_sp.at[pl.ds(row0, RPT)], stage_v)
    pltpu.sync_copy(stage_v, dout.at[pl.ds(obase, RPT)])


def _leaky(v):
    return jnp.where(v > 0, v, 0.2 * v)


def _t1_body(deg2_ref, x_ref, y_ref):
    d = deg2_ref[0] + deg2_ref[1] + 1.0
    dinv = (1.0 / jnp.sqrt(d))[:N]
    y_ref[...] = x_ref[...] * dinv


def _t2_body(zp_ref, x_ref, deg2_ref, t_ref, Wo_ref, bo_ref, Wt_ref, bt_ref,
             a1s_ref, a2s_ref, a1d_ref, a2d_ref,
             ro_ref, rt_ref, sA_ref, sD_ref, gmax_ref):
    d = deg2_ref[0] + deg2_ref[1] + 1.0
    dinv = (1.0 / jnp.sqrt(d))[:N]
    zsum = zp_ref[0, :N] + zp_ref[1, :N]
    x = x_ref[...]
    ax = dinv * zsum + (dinv * dinv) * x
    ro = jnp.maximum(
        jnp.dot(ax, Wo_ref[...], preferred_element_type=jnp.float32)
        + bo_ref[...], 0.0)
    rt = jnp.maximum(
        jnp.dot(ax, Wt_ref[...], preferred_element_type=jnp.float32)
        + bt_ref[...], 0.0)
    ro_ref[...] = ro
    rt_ref[...] = rt
    s_src = (jnp.dot(ro, a1s_ref[...], preferred_element_type=jnp.float32)
             + jnp.dot(rt, a2s_ref[...], preferred_element_type=jnp.float32))
    s_dst = (jnp.dot(ro, a1d_ref[...], preferred_element_type=jnp.float32)
             + jnp.dot(rt, a2d_ref[...], preferred_element_type=jnp.float32))
    gmax_ref[...] = jnp.broadcast_to(jnp.max(s_src), (1, H))
    sA_ref[...] = jnp.where(t_ref[...] > 0, s_src, -1e30)
    sD_ref[...] = s_dst


def _mlp(v, W1, b1, W2, b2, W3, b3):
    h1 = _leaky(jnp.dot(v, W1, preferred_element_type=jnp.float32) + b1)
    h2 = _leaky(jnp.dot(h1, W2, preferred_element_type=jnp.float32) + b2)
    return jnp.dot(h2, W3, preferred_element_type=jnp.float32) + b3


def _t3_body(ro_ref, rt_ref, hp_ref, dp_ref, t_ref,
             We1_ref, We2_ref, be_ref,
             p1_refs, p0_refs, d_refs,
             predt_ref, pred_ref, zrep_ref):
    den = dp_ref[0] + dp_ref[1] + 1e-16
    h = (hp_ref[0] + hp_ref[1]) / den
    ro = ro_ref[...]
    rt = rt_ref[...]
    zrep = (jnp.dot(ro, We1_ref[...], preferred_element_type=jnp.float32)
            + jnp.dot(h, We2_ref[...], preferred_element_type=jnp.float32)
            + be_ref[...])
    zrep_ref[...] = zrep
    pred1 = _mlp(zrep, *[r[...] for r in p1_refs])
    pred0 = _mlp(zrep, *[r[...] for r in p0_refs])
    pred_ref[...] = jnp.where(t_ref[...] > 0, pred1, pred0)
    predt_ref[...] = jax.nn.sigmoid(_mlp(rt, *[r[...] for r in d_refs]))


def _t3_flat(ro, rt, hp, dp, t2,
             We1, We2, be,
             p1_W1, p1_b1, p1_W2, p1_b2, p1_W3, p1_b3,
             p0_W1, p0_b1, p0_W2, p0_b2, p0_W3, p0_b3,
             d_W1, d_b1, d_W2, d_b2, d_W3, d_b3,
             predt_ref, pred_ref, zrep_ref):
    _t3_body(ro, rt, hp, dp, t2, We1, We2, be,
             (p1_W1, p1_b1, p1_W2, p1_b2, p1_W3, p1_b3),
             (p0_W1, p0_b1, p0_W2, p0_b2, p0_W3, p0_b3),
             (d_W1, d_b1, d_W2, d_b2, d_W3, d_b3),
             predt_ref, pred_ref, zrep_ref)


def kernel(x, t, z, edge_index, W_gcn_o, b_gcn_o, W_gcn_t, b_gcn_t, a_src,
           a_dst, W_enc, b_enc, p1_W1, p1_b1, p1_W2, p1_b2, p1_W3, p1_b3,
           p0_W1, p0_b1, p0_W2, p0_b2, p0_W3, p0_b3, d_W1, d_b1, d_W2, d_b2,
           d_W3, d_b3):
    f32 = jnp.float32
    src = edge_index[0].astype(jnp.int32)
    dst = edge_index[1].astype(jnp.int32)
    npad = EPAD - E
    srcE = jnp.concatenate(
        [src, jnp.arange(npad, dtype=jnp.int32) % 1024]).reshape(NW, EPW)
    dstE = jnp.concatenate(
        [dst, N + (jnp.arange(npad, dtype=jnp.int32) % (NACC - N))]
    ).reshape(NW, EPW)

    deg_parts = _s1_deg(dstE)
    deg2 = deg_parts.reshape(NC, NACC, 1)

    y = pl.pallas_call(
        _t1_body,
        out_shape=jax.ShapeDtypeStruct((N, H), f32),
    )(deg2, x)

    z_parts = _s2_agg(srcE, dstE, y)

    t2d = t.astype(f32).reshape(N, 1)
    ro, rt, sA2, sD2, gmax2 = pl.pallas_call(
        _t2_body,
        out_shape=(
            jax.ShapeDtypeStruct((N, H), f32),
            jax.ShapeDtypeStruct((N, H), f32),
            jax.ShapeDtypeStruct((N, 1), f32),
            jax.ShapeDtypeStruct((N, 1), f32),
            jax.ShapeDtypeStruct((1, H), f32),
        ),
    )(z_parts, x, deg2, t2d,
      W_gcn_o, b_gcn_o.reshape(1, H), W_gcn_t, b_gcn_t.reshape(1, H),
      a_src[:H].reshape(H, 1), a_src[H:].reshape(H, 1),
      a_dst[:H].reshape(H, 1), a_dst[H:].reshape(H, 1))

    sA = sA2.reshape(N)
    sD = sD2.reshape(N)
    gmax16 = gmax2[0, :16]
    hacc_parts, den_parts = _s3_attn(srcE, dstE, rt, sA, sD, gmax16)

    R = 2048
    _w = pl.BlockSpec((H, H), lambda i: (0, 0))
    _b = pl.BlockSpec((1, H), lambda i: (0, 0))
    _w3 = pl.BlockSpec((H, 1), lambda i: (0, 0))
    _b3 = pl.BlockSpec((1, 1), lambda i: (0, 0))
    _rows = pl.BlockSpec((R, H), lambda i: (i, 0))
    _col = pl.BlockSpec((R, 1), lambda i: (i, 0))
    pred_t, pred, zrep = pl.pallas_call(
        _t3_flat,
        grid=(NACC // R,),
        in_specs=[
            _rows, _rows,
            pl.BlockSpec((NC, R, H), lambda i: (0, i, 0)),
            pl.BlockSpec((NC, R, 1), lambda i: (0, i, 0)),
            _col,
            _w, _w, _b,
            _w, _b, _w, _b, _w3, _b3,
            _w, _b, _w, _b, _w3, _b3,
            _w, _b, _w, _b, _w3, _b3,
        ],
        out_specs=(_col, _col, _rows),
        out_shape=(
            jax.ShapeDtypeStruct((N, 1), f32),
            jax.ShapeDtypeStruct((N, 1), f32),
            jax.ShapeDtypeStruct((N, H), f32),
        ),
    )(ro, rt, hacc_parts, den_parts.reshape(NC, NACC, 1), t2d,
      W_enc[:H], W_enc[H:], b_enc.reshape(1, H),
      p1_W1, p1_b1.reshape(1, H), p1_W2, p1_b2.reshape(1, H),
      p1_W3, p1_b3.reshape(1, 1),
      p0_W1, p0_b1.reshape(1, H), p0_W2, p0_b2.reshape(1, H),
      p0_W3, p0_b3.reshape(1, 1),
      d_W1, d_b1.reshape(1, H), d_W2, d_b2.reshape(1, H),
      d_W3, d_b3.reshape(1, 1))
    return (pred_t, pred, zrep)
